# baseline (speedup 1.0000x reference)
import functools
import jax
import jax.numpy as jnp
from jax import lax
from jax.experimental import pallas as pl
from jax.experimental.pallas import tpu as pltpu
from jax.experimental.pallas import tpu_sc as plsc

B, L, D = 8, 256, 1024
NL = 24
E = 96
K = 2
H = 128
CAP = 128
NCLS = 2
N = B * L
EPAD = 128
NEG = -1e30


def _router_body(last_r, wd_r, wg_r, bd_r, row_r, gate_r, batch_r,
                 x_r, c_r):
    moe = jnp.dot(last_r[...], wd_r[...], preferred_element_type=jnp.float32)
    moe = moe + bd_r[0:1, :]
    clean = jnp.dot(moe, wg_r[...], preferred_element_type=jnp.float32)

    lane = lax.broadcasted_iota(jnp.int32, (N, EPAD), 1)
    cleanm = jnp.where(lane < E, clean, NEG)

    m1 = jnp.max(cleanm, axis=1, keepdims=True)
    a1 = jnp.min(jnp.where(cleanm == m1, lane, EPAD), axis=1, keepdims=True)
    clean2 = jnp.where(lane == a1, NEG, cleanm)
    m2 = jnp.max(clean2, axis=1, keepdims=True)
    a2 = jnp.min(jnp.where(clean2 == m2, lane, EPAD), axis=1, keepdims=True)
    g1 = 1.0 / (1.0 + jnp.exp(m2 - m1))
    g2 = 1.0 - g1

    oh1 = (lane == a1).astype(jnp.float32)
    oh2 = (lane == a2).astype(jnp.float32)
    X = oh1 + oh2

    BLK = 256
    r_i = lax.broadcasted_iota(jnp.int32, (BLK, BLK), 0)
    c_i = lax.broadcasted_iota(jnp.int32, (BLK, BLK), 1)
    tri = (r_i > c_i).astype(jnp.float32)

    x_r[...] = X

    def step(i, run):
        xb = x_r[pl.ds(i * BLK, BLK), :]
        cb = jnp.dot(tri, xb, preferred_element_type=jnp.float32) + run
        c_r[pl.ds(i * BLK, BLK), :] = cb
        return run + jnp.sum(xb, axis=0, keepdims=True)

    run0 = jnp.zeros((1, EPAD), jnp.float32)
    lax.fori_loop(0, N // BLK, step, run0)
    C = c_r[...]

    pos1 = jnp.sum(C * oh1, axis=1, keepdims=True)
    pos2 = jnp.sum(C * oh2, axis=1, keepdims=True)
    keep1 = pos1 < CAP
    keep2 = pos2 < CAP
    posl = lax.broadcasted_iota(jnp.int32, (N, CAP), 1).astype(jnp.float32)
    ohp1 = jnp.where((posl == pos1) & keep1, 1.0, 0.0)
    ohp2 = jnp.where((posl == pos2) & keep2, 1.0, 0.0)

    tok = lax.broadcasted_iota(jnp.int32, (N, 1), 0).astype(jnp.float32)
    batch = jnp.floor(tok * (1.0 / L))
    row1 = jnp.floor(a1.astype(jnp.float32) * (1.0 / (E // NL))) * N + tok
    row2 = jnp.floor(a2.astype(jnp.float32) * (1.0 / (E // NL))) * N + tok

    dn = (((0,), (0,)), ((), ()))

    def tab(oh, ohp, val):
        return lax.dot_general(oh, ohp * val, dn,
                               preferred_element_type=jnp.float32,
                               precision=lax.Precision.HIGHEST)

    row_r[...] = tab(oh1, ohp1, row1) + tab(oh2, ohp2, row2)
    gate_r[...] = tab(oh1, ohp1, g1) + tab(oh2, ohp2, g2)
    batch_r[...] = tab(oh1, ohp1, batch) + tab(oh2, ohp2, batch)


def _router_call(last2d, wd_sl, wg_pad, bd2d):
    out = [
        jax.ShapeDtypeStruct((EPAD, CAP), jnp.float32),
        jax.ShapeDtypeStruct((EPAD, CAP), jnp.float32),
        jax.ShapeDtypeStruct((EPAD, CAP), jnp.float32),
    ]
    return pl.pallas_call(
        _router_body,
        out_shape=out,
        scratch_shapes=[
            pltpu.VMEM((N, EPAD), jnp.float32),
            pltpu.VMEM((N, EPAD), jnp.float32),
        ],
    )(last2d, wd_sl, wg_pad, bd2d)


_SC_CHUNK = 24
_SC_NBUF = 4


def _sc_gather(hid2d, idx):
    info = plsc.get_sparse_core_info()
    nw = info.num_cores * info.num_subcores
    nrows = idx.shape[0]
    per_w = nrows // nw
    nchunk = per_w // _SC_CHUNK
    mesh = plsc.VectorSubcoreMesh(core_axis_name="c", subcore_axis_name="s")

    @functools.partial(
        pl.kernel,
        out_type=jax.ShapeDtypeStruct((nrows, D), jnp.float32),
        mesh=mesh,
        scratch_types=[
            pltpu.VMEM((per_w,), jnp.int32),
        ] + [pltpu.VMEM((_SC_CHUNK, D), jnp.float32)] * _SC_NBUF
          + [pltpu.SemaphoreType.DMA] * (2 * _SC_NBUF),
        compiler_params=pltpu.CompilerParams(use_tc_tiling_on_sc=True),
    )
    def gather_k(hid_hbm, idx_hbm, out_hbm, idx_v, *bufsem):
        bufs = bufsem[:_SC_NBUF]
        gsem = bufsem[_SC_NBUF:2 * _SC_NBUF]
        osem = bufsem[2 * _SC_NBUF:]
        wid = lax.axis_index("s") * info.num_cores + lax.axis_index("c")
        base = wid * per_w
        pltpu.sync_copy(idx_hbm.at[pl.ds(base, per_w)], idx_v)

        def gather_cp(c):
            b = c % _SC_NBUF
            return pltpu.make_async_copy(
                hid_hbm.at[idx_v.at[pl.ds(c * _SC_CHUNK, _SC_CHUNK)]],
                bufs[b], gsem[b])

        def out_cp(c):
            b = c % _SC_NBUF
            return pltpu.make_async_copy(
                bufs[b], out_hbm.at[pl.ds(base + c * _SC_CHUNK, _SC_CHUNK)],
                osem[b])

        for c in range(_SC_NBUF):
            gather_cp(c).start()
        for c in range(nchunk):
            gather_cp(c).wait()
            out_cp(c).start()
            n = c + _SC_NBUF
            if n < nchunk:
                out_cp(n - _SC_NBUF).wait()
                gather_cp(n).start()
        for c in range(nchunk - _SC_NBUF, nchunk):
            out_cp(c).wait()

    return gather_k(hid2d, idx)


def _expert_body(feats_r, w1_r, b1_r, w2_r, b2_r, wc_r, gate_r, batch_r,
                 bc_r, out_r, acc_r):
    e = pl.program_id(0)
    h = jnp.dot(feats_r[0], w1_r[0], preferred_element_type=jnp.float32)
    h = jnp.maximum(h + b1_r[0], 0.0)
    Ve = jnp.dot(w2_r[0], wc_r[...], preferred_element_type=jnp.float32)
    y = jnp.dot(h, Ve, preferred_element_type=jnp.float32)
    y = y + jnp.dot(b2_r[0], wc_r[...], preferred_element_type=jnp.float32)

    srow = lax.broadcasted_iota(jnp.int32, (B, CAP), 0).astype(jnp.float32)
    M = jnp.where(srow == batch_r[0], gate_r[0], 0.0)
    contrib = jnp.dot(M, y, preferred_element_type=jnp.float32)

    @pl.when(e == 0)
    def _init():
        acc_r[...] = jnp.zeros_like(acc_r)

    acc_r[...] += contrib

    @pl.when(e == E - 1)
    def _fini():
        out_r[...] = acc_r[...] * (1.0 / L) + bc_r[...]


def _expert_call(feats3, W1, b1_3, W2, b2_3, wc_pad, gate3, batch3, bc2d):
    grid = (E,)
    specs = [
        pl.BlockSpec((1, CAP, D), lambda e: (e, 0, 0)),
        pl.BlockSpec((1, D, H), lambda e: (e, 0, 0)),
        pl.BlockSpec((1, 1, H), lambda e: (e, 0, 0)),
        pl.BlockSpec((1, H, D), lambda e: (e, 0, 0)),
        pl.BlockSpec((1, 1, D), lambda e: (e, 0, 0)),
        pl.BlockSpec((D, EPAD), lambda e: (0, 0)),
        pl.BlockSpec((1, 1, CAP), lambda e: (e, 0, 0)),
        pl.BlockSpec((1, 1, CAP), lambda e: (e, 0, 0)),
        pl.BlockSpec((B, EPAD), lambda e: (0, 0)),
    ]
    return pl.pallas_call(
        _expert_body,
        grid=grid,
        in_specs=specs,
        out_specs=pl.BlockSpec((B, EPAD), lambda e: (0, 0)),
        out_shape=jax.ShapeDtypeStruct((B, EPAD), jnp.float32),
        scratch_shapes=[pltpu.VMEM((B, EPAD), jnp.float32)],
    )(feats3, W1, b1_3, W2, b2_3, wc_pad, gate3, batch3, bc2d)


def kernel(last, hidden_all, Wd, bd, w_gate, W1, b1, W2, b2, Wc, bc):
    last2d = last.reshape(N, D)
    wd_sl = Wd[:D]
    wg_pad = jnp.pad(w_gate, ((0, 0), (0, EPAD - E)))
    bd2d = jnp.broadcast_to(bd[None, :], (8, D))

    row_t, gate_t, batch_t = _router_call(last2d, wd_sl, wg_pad, bd2d)

    idx = jnp.rint(row_t[:E]).astype(jnp.int32).reshape(E * CAP)
    hid2d = hidden_all.reshape(NL * N, D)
    feats = _sc_gather(hid2d, idx)

    feats3 = feats.reshape(E, CAP, D)
    gate3 = gate_t[:E].reshape(E, 1, CAP)
    batch3 = batch_t[:E].reshape(E, 1, CAP)
    b1_3 = b1.reshape(E, 1, H)
    b2_3 = b2.reshape(E, 1, D)
    wc_pad = jnp.pad(Wc, ((0, 0), (0, EPAD - NCLS)))
    bc2d = jnp.broadcast_to(
        jnp.pad(bc, (0, EPAD - NCLS))[None, :], (B, EPAD))

    out = _expert_call(feats3, W1, b1_3, W2, b2_3, wc_pad, gate3, batch3,
                       bc2d)
    return out[:, :NCLS]


# speedup vs baseline: 3.6352x; 3.6352x over previous
import jax
import jax.numpy as jnp
from jax import lax
from jax.experimental import pallas as pl
from jax.experimental.pallas import tpu as pltpu

B, L, D = 8, 256, 1024
NL = 24
E = 96
K = 2
H = 128
CAP = 128
NCLS = 2
N = B * L
EG = E // NL
EPAD = 128
NEG = -1e30


def _router_body(last_r, wd_r, wg_r, bd_r, tok_r, gate_r, batch_r,
                 x_r, c_r):
    moe = jnp.dot(last_r[...], wd_r[...], preferred_element_type=jnp.float32)
    moe = moe + bd_r[0:1, :]
    clean = jnp.dot(moe, wg_r[...], preferred_element_type=jnp.float32)

    lane = lax.broadcasted_iota(jnp.int32, (N, EPAD), 1)
    cleanm = jnp.where(lane < E, clean, NEG)

    m1 = jnp.max(cleanm, axis=1, keepdims=True)
    a1 = jnp.min(jnp.where(cleanm == m1, lane, EPAD), axis=1, keepdims=True)
    clean2 = jnp.where(lane == a1, NEG, cleanm)
    m2 = jnp.max(clean2, axis=1, keepdims=True)
    a2 = jnp.min(jnp.where(clean2 == m2, lane, EPAD), axis=1, keepdims=True)
    g1 = 1.0 / (1.0 + jnp.exp(m2 - m1))
    g2 = 1.0 - g1

    oh1 = (lane == a1).astype(jnp.float32)
    oh2 = (lane == a2).astype(jnp.float32)
    X = oh1 + oh2

    BLK = 256
    r_i = lax.broadcasted_iota(jnp.int32, (BLK, BLK), 0)
    c_i = lax.broadcasted_iota(jnp.int32, (BLK, BLK), 1)
    tri = (r_i > c_i).astype(jnp.float32)

    x_r[...] = X

    def step(i, run):
        xb = x_r[pl.ds(i * BLK, BLK), :]
        cb = jnp.dot(tri, xb, preferred_element_type=jnp.float32) + run
        c_r[pl.ds(i * BLK, BLK), :] = cb
        return run + jnp.sum(xb, axis=0, keepdims=True)

    run0 = jnp.zeros((1, EPAD), jnp.float32)
    lax.fori_loop(0, N // BLK, step, run0)
    C = c_r[...]

    pos1 = jnp.sum(C * oh1, axis=1, keepdims=True)
    pos2 = jnp.sum(C * oh2, axis=1, keepdims=True)
    keep1 = pos1 < CAP
    keep2 = pos2 < CAP
    posl = lax.broadcasted_iota(jnp.int32, (N, CAP), 1).astype(jnp.float32)
    ohp1 = jnp.where((posl == pos1) & keep1, 1.0, 0.0)
    ohp2 = jnp.where((posl == pos2) & keep2, 1.0, 0.0)

    tok = lax.broadcasted_iota(jnp.int32, (N, 1), 0).astype(jnp.float32)
    batch = jnp.floor(tok * (1.0 / L))

    dn = (((0,), (0,)), ((), ()))

    def tab(oh, ohp, val):
        return lax.dot_general(oh, ohp * val, dn,
                               preferred_element_type=jnp.float32,
                               precision=lax.Precision.HIGHEST)

    tok_r[...] = tab(oh1, ohp1, tok) + tab(oh2, ohp2, tok)
    gate_r[...] = tab(oh1, ohp1, g1) + tab(oh2, ohp2, g2)
    batch_r[...] = tab(oh1, ohp1, batch) + tab(oh2, ohp2, batch)


def _router_call(last2d, wd_sl, wg_pad, bd2d):
    out = [
        jax.ShapeDtypeStruct((EPAD, CAP), jnp.float32),
        jax.ShapeDtypeStruct((EPAD, CAP), jnp.float32),
        jax.ShapeDtypeStruct((EPAD, CAP), jnp.float32),
    ]
    return pl.pallas_call(
        _router_body,
        out_shape=out,
        scratch_shapes=[
            pltpu.VMEM((N, EPAD), jnp.float32),
            pltpu.VMEM((N, EPAD), jnp.float32),
        ],
    )(last2d, wd_sl, wg_pad, bd2d)


def _expert_body(hid_r, w1_r, b1_r, w2_r, b2_r, wc_r, tok_r, gate_r,
                 batch_r, bc_r, out_r, acc_r):
    li = pl.program_id(0)
    hid_l = hid_r[0]

    @pl.when(li == 0)
    def _init():
        acc_r[...] = jnp.zeros_like(acc_r)

    t_iota = lax.broadcasted_iota(jnp.int32, (N, CAP), 0).astype(jnp.float32)
    srow = lax.broadcasted_iota(jnp.int32, (B, CAP), 0).astype(jnp.float32)
    dn = (((0,), (0,)), ((), ()))
    acc = acc_r[...]
    for q in range(EG):
        ohT = jnp.where(t_iota == tok_r[0, q], 1.0, 0.0)
        feats = lax.dot_general(ohT, hid_l, dn,
                                preferred_element_type=jnp.float32)
        h = jnp.dot(feats, w1_r[0, q], preferred_element_type=jnp.float32)
        h = jnp.maximum(h + b1_r[0, q], 0.0)
        Ve = jnp.dot(w2_r[0, q], wc_r[...],
                     preferred_element_type=jnp.float32)
        y = jnp.dot(h, Ve, preferred_element_type=jnp.float32)
        y = y + jnp.dot(b2_r[0, q], wc_r[...],
                        preferred_element_type=jnp.float32)
        M = jnp.where(srow == batch_r[0, q], gate_r[0, q], 0.0)
        acc = acc + jnp.dot(M, y, preferred_element_type=jnp.float32)
    acc_r[...] = acc

    @pl.when(li == NL - 1)
    def _fini():
        out_r[...] = acc_r[...] * (1.0 / L) + bc_r[...]


def _expert_call(hid3, W1_4, b1_4, W2_4, b2_4, wc_pad, tok4, gate4, batch4,
                 bc2d):
    grid = (NL,)
    specs = [
        pl.BlockSpec((1, N, D), lambda l: (l, 0, 0)),
        pl.BlockSpec((1, EG, D, H), lambda l: (l, 0, 0, 0)),
        pl.BlockSpec((1, EG, 1, H), lambda l: (l, 0, 0, 0)),
        pl.BlockSpec((1, EG, H, D), lambda l: (l, 0, 0, 0)),
        pl.BlockSpec((1, EG, 1, D), lambda l: (l, 0, 0, 0)),
        pl.BlockSpec((D, EPAD), lambda l: (0, 0)),
        pl.BlockSpec((1, EG, 1, CAP), lambda l: (l, 0, 0, 0)),
        pl.BlockSpec((1, EG, 1, CAP), lambda l: (l, 0, 0, 0)),
        pl.BlockSpec((1, EG, 1, CAP), lambda l: (l, 0, 0, 0)),
        pl.BlockSpec((B, EPAD), lambda l: (0, 0)),
    ]
    return pl.pallas_call(
        _expert_body,
        grid=grid,
        in_specs=specs,
        out_specs=pl.BlockSpec((B, EPAD), lambda l: (0, 0)),
        out_shape=jax.ShapeDtypeStruct((B, EPAD), jnp.float32),
        scratch_shapes=[pltpu.VMEM((B, EPAD), jnp.float32)],
    )(hid3, W1_4, b1_4, W2_4, b2_4, wc_pad, tok4, gate4, batch4, bc2d)


def kernel(last, hidden_all, Wd, bd, w_gate, W1, b1, W2, b2, Wc, bc):
    last2d = last.reshape(N, D)
    wd_sl = Wd[:D]
    wg_pad = jnp.pad(w_gate, ((0, 0), (0, EPAD - E)))
    bd2d = jnp.broadcast_to(bd[None, :], (8, D))

    tok_t, gate_t, batch_t = _router_call(last2d, wd_sl, wg_pad, bd2d)

    hid3 = hidden_all.reshape(NL, N, D)
    tok4 = tok_t[:E].reshape(NL, EG, 1, CAP)
    gate4 = gate_t[:E].reshape(NL, EG, 1, CAP)
    batch4 = batch_t[:E].reshape(NL, EG, 1, CAP)
    W1_4 = W1.reshape(NL, EG, D, H)
    b1_4 = b1.reshape(NL, EG, 1, H)
    W2_4 = W2.reshape(NL, EG, H, D)
    b2_4 = b2.reshape(NL, EG, 1, D)
    wc_pad = jnp.pad(Wc, ((0, 0), (0, EPAD - NCLS)))
    bc2d = jnp.broadcast_to(
        jnp.pad(bc, (0, EPAD - NCLS))[None, :], (B, EPAD))

    out = _expert_call(hid3, W1_4, b1_4, W2_4, b2_4, wc_pad, tok4, gate4,
                       batch4, bc2d)
    return out[:, :NCLS]


# speedup vs baseline: 3.8465x; 1.0581x over previous
import jax
import jax.numpy as jnp
from jax import lax
from jax.experimental import pallas as pl
from jax.experimental.pallas import tpu as pltpu

B, L, D = 8, 256, 1024
NL = 24
E = 96
K = 2
H = 128
CAP = 128
NCLS = 2
N = B * L
EG = E // NL
EPAD = 128
NEG = -1e30


def _router_body(last_r, wd_r, wg_r, bd_r, tok_r, gate_r, batch_r,
                 x_r, c_r):
    last2d = last_r[...].reshape(N, D)
    moe = jnp.dot(last2d, wd_r[...], preferred_element_type=jnp.float32)
    moe = moe + bd_r[0:1, :]
    clean = jnp.dot(moe, wg_r[...], preferred_element_type=jnp.float32)

    lane = lax.broadcasted_iota(jnp.int32, (N, EPAD), 1)
    cleanm = jnp.where(lane < E, clean, NEG)

    m1 = jnp.max(cleanm, axis=1, keepdims=True)
    a1 = jnp.min(jnp.where(cleanm == m1, lane, EPAD), axis=1, keepdims=True)
    clean2 = jnp.where(lane == a1, NEG, cleanm)
    m2 = jnp.max(clean2, axis=1, keepdims=True)
    a2 = jnp.min(jnp.where(clean2 == m2, lane, EPAD), axis=1, keepdims=True)
    g1 = 1.0 / (1.0 + jnp.exp(m2 - m1))
    g2 = 1.0 - g1

    oh1 = (lane == a1).astype(jnp.float32)
    oh2 = (lane == a2).astype(jnp.float32)
    X = oh1 + oh2

    BLK = 256
    r_i = lax.broadcasted_iota(jnp.int32, (BLK, BLK), 0)
    c_i = lax.broadcasted_iota(jnp.int32, (BLK, BLK), 1)
    tri = (r_i > c_i).astype(jnp.float32)

    x_r[...] = X

    def step(i, run):
        xb = x_r[pl.ds(i * BLK, BLK), :]
        cb = jnp.dot(tri, xb, preferred_element_type=jnp.float32) + run
        c_r[pl.ds(i * BLK, BLK), :] = cb
        return run + jnp.sum(xb, axis=0, keepdims=True)

    run0 = jnp.zeros((1, EPAD), jnp.float32)
    lax.fori_loop(0, N // BLK, step, run0)
    C = c_r[...]

    pos1 = jnp.sum(C * oh1, axis=1, keepdims=True)
    pos2 = jnp.sum(C * oh2, axis=1, keepdims=True)
    keep1 = pos1 < CAP
    keep2 = pos2 < CAP
    posl = lax.broadcasted_iota(jnp.int32, (N, CAP), 1).astype(jnp.float32)
    ohp1 = jnp.where((posl == pos1) & keep1, 1.0, 0.0)
    ohp2 = jnp.where((posl == pos2) & keep2, 1.0, 0.0)

    tok = lax.broadcasted_iota(jnp.int32, (N, 1), 0).astype(jnp.float32)
    batch = jnp.floor(tok * (1.0 / L))

    dn = (((0,), (0,)), ((), ()))

    def tab(oh, ohp, val):
        return lax.dot_general(oh, ohp * val, dn,
                               preferred_element_type=jnp.float32,
                               precision=lax.Precision.HIGHEST)

    tok_r[...] = (tab(oh1, ohp1, tok) + tab(oh2, ohp2, tok))[:E]
    gate_r[...] = (tab(oh1, ohp1, g1) + tab(oh2, ohp2, g2))[:E]
    batch_r[...] = (tab(oh1, ohp1, batch) + tab(oh2, ohp2, batch))[:E]


def _router_call(last, Wd, wg_pad, bd2d):
    out = [
        jax.ShapeDtypeStruct((E, CAP), jnp.float32),
        jax.ShapeDtypeStruct((E, CAP), jnp.float32),
        jax.ShapeDtypeStruct((E, CAP), jnp.float32),
    ]
    specs = [
        pl.BlockSpec((B, L, D), lambda i: (0, 0, 0)),
        pl.BlockSpec((D, D), lambda i: (0, 0)),
        pl.BlockSpec((D, EPAD), lambda i: (0, 0)),
        pl.BlockSpec((B, D), lambda i: (0, 0)),
    ]
    return pl.pallas_call(
        _router_body,
        grid=(1,),
        in_specs=specs,
        out_specs=[pl.BlockSpec((E, CAP), lambda i: (0, 0))] * 3,
        out_shape=out,
        scratch_shapes=[
            pltpu.VMEM((N, EPAD), jnp.float32),
            pltpu.VMEM((N, EPAD), jnp.float32),
        ],
    )(last, Wd, wg_pad, bd2d)


def _expert_body(hid_r, w1_r, b1_r, w2_r, b2_r, wc_r, tok_r, gate_r,
                 batch_r, bc_r, out_r, acc_r):
    li = pl.program_id(0)
    hid_l = hid_r[0]

    @pl.when(li == 0)
    def _init():
        acc_r[...] = jnp.zeros_like(acc_r)

    t_iota = lax.broadcasted_iota(jnp.int32, (N, CAP), 0).astype(jnp.float32)
    srow = lax.broadcasted_iota(jnp.int32, (B, CAP), 0).astype(jnp.float32)
    dn = (((0,), (0,)), ((), ()))
    acc = acc_r[...]
    for q in range(EG):
        ohT = jnp.where(t_iota == tok_r[0, q], 1.0, 0.0)
        feats = lax.dot_general(ohT, hid_l, dn,
                                preferred_element_type=jnp.float32)
        h = jnp.dot(feats, w1_r[0, q], preferred_element_type=jnp.float32)
        h = jnp.maximum(h + b1_r[0, q], 0.0)
        Ve = jnp.dot(w2_r[0, q], wc_r[...],
                     preferred_element_type=jnp.float32)
        y = jnp.dot(h, Ve, preferred_element_type=jnp.float32)
        y = y + jnp.dot(b2_r[0, q], wc_r[...],
                        preferred_element_type=jnp.float32)
        M = jnp.where(srow == batch_r[0, q], gate_r[0, q], 0.0)
        acc = acc + jnp.dot(M, y, preferred_element_type=jnp.float32)
    acc_r[...] = acc

    @pl.when(li == NL - 1)
    def _fini():
        out_r[...] = acc_r[...] * (1.0 / L) + bc_r[...]


def _expert_call(hid3, W1_4, b1_4, W2_4, b2_4, wc_pad, tok4, gate4, batch4,
                 bc2d):
    grid = (NL,)
    specs = [
        pl.BlockSpec((1, N, D), lambda l: (l, 0, 0)),
        pl.BlockSpec((1, EG, D, H), lambda l: (l, 0, 0, 0)),
        pl.BlockSpec((1, EG, 1, H), lambda l: (l, 0, 0, 0)),
        pl.BlockSpec((1, EG, H, D), lambda l: (l, 0, 0, 0)),
        pl.BlockSpec((1, EG, 1, D), lambda l: (l, 0, 0, 0)),
        pl.BlockSpec((D, EPAD), lambda l: (0, 0)),
        pl.BlockSpec((1, EG, 1, CAP), lambda l: (l, 0, 0, 0)),
        pl.BlockSpec((1, EG, 1, CAP), lambda l: (l, 0, 0, 0)),
        pl.BlockSpec((1, EG, 1, CAP), lambda l: (l, 0, 0, 0)),
        pl.BlockSpec((B, EPAD), lambda l: (0, 0)),
    ]
    return pl.pallas_call(
        _expert_body,
        grid=grid,
        in_specs=specs,
        out_specs=pl.BlockSpec((B, EPAD), lambda l: (0, 0)),
        out_shape=jax.ShapeDtypeStruct((B, EPAD), jnp.float32),
        scratch_shapes=[pltpu.VMEM((B, EPAD), jnp.float32)],
    )(hid3, W1_4, b1_4, W2_4, b2_4, wc_pad, tok4, gate4, batch4, bc2d)


def kernel(last, hidden_all, Wd, bd, w_gate, W1, b1, W2, b2, Wc, bc):
    wg_pad = jnp.pad(w_gate, ((0, 0), (0, EPAD - E)))
    bd2d = jnp.broadcast_to(bd[None, :], (B, D))

    tok_t, gate_t, batch_t = _router_call(last, Wd, wg_pad, bd2d)

    hid3 = hidden_all.reshape(NL, N, D)
    tok4 = tok_t.reshape(NL, EG, 1, CAP)
    gate4 = gate_t.reshape(NL, EG, 1, CAP)
    batch4 = batch_t.reshape(NL, EG, 1, CAP)
    W1_4 = W1.reshape(NL, EG, D, H)
    b1_4 = b1.reshape(NL, EG, 1, H)
    W2_4 = W2.reshape(NL, EG, H, D)
    b2_4 = b2.reshape(NL, EG, 1, D)
    wc_pad = jnp.pad(Wc, ((0, 0), (0, EPAD - NCLS)))
    bc2d = jnp.broadcast_to(
        jnp.pad(bc, (0, EPAD - NCLS))[None, :], (B, EPAD))

    out = _expert_call(hid3, W1_4, b1_4, W2_4, b2_4, wc_pad, tok4, gate4,
                       batch4, bc2d)
    return out[:, :NCLS]
